# Initial kernel scaffold; baseline (speedup 1.0000x reference)
import jax
import jax.numpy as jnp
from jax.experimental import pallas as pl


def kernel(x, embeddings, W_ih, W_hh, b_ih, b_hh, W_fc, b_fc):
    raise NotImplementedError("write your pallas kernel here")


# speedup vs baseline: 4.2331x; 4.2331x over previous
import functools

import jax
import jax.numpy as jnp
from jax import lax
from jax.experimental import pallas as pl
from jax.experimental.pallas import tpu as pltpu
from jax.experimental.pallas import tpu_sc as plsc

_NC = 2
_NS = 16
_NW = _NC * _NS
_GATHER_CHUNK = 128


@functools.lru_cache(maxsize=None)
def _make_sc_gather(vocab, emb, n_idx):
    rows = n_idx // _GATHER_CHUNK
    rpw = rows // _NW
    mesh = plsc.VectorSubcoreMesh(core_axis_name="c", subcore_axis_name="s")

    @functools.partial(
        pl.kernel,
        mesh=mesh,
        out_type=jax.ShapeDtypeStruct((n_idx, emb), jnp.float32),
        scratch_types=[
            pltpu.VMEM((rpw, _GATHER_CHUNK), jnp.int32),
            pltpu.VMEM((_GATHER_CHUNK, emb), jnp.float32),
            pltpu.SemaphoreType.DMA,
        ],
        compiler_params=pltpu.CompilerParams(use_tc_tiling_on_sc=False),
    )
    def gather_kernel(table_hbm, idx_hbm, out_hbm, idx_v, rows_v, sem):
        wid = lax.axis_index("s") * _NC + lax.axis_index("c")
        base = wid * rpw
        pltpu.sync_copy(idx_hbm.at[pl.ds(base, rpw)], idx_v)

        def body(j, carry):
            pltpu.async_copy(table_hbm.at[idx_v.at[j]], rows_v, sem).wait()
            pltpu.sync_copy(
                rows_v, out_hbm.at[pl.ds((base + j) * _GATHER_CHUNK, _GATHER_CHUNK)]
            )
            return carry

        lax.fori_loop(0, rpw, body, 0)

    return gather_kernel


@functools.lru_cache(maxsize=None)
def _make_rnn_fc(t_steps, batch, emb, hid, out_dim):
    def rnn_kernel(emb_ref, wih_ref, whh_ref, b_ref, wfc_ref, bfc_ref,
                   out_ref, h_ref):
        t = pl.program_id(0)

        @pl.when(t == 0)
        def _():
            h_ref[...] = jnp.zeros_like(h_ref)

        x = emb_ref[0]
        h = jnp.tanh(
            jnp.dot(x, wih_ref[...], preferred_element_type=jnp.float32)
            + jnp.dot(h_ref[...], whh_ref[...], preferred_element_type=jnp.float32)
            + b_ref[...]
        )
        h_ref[...] = h

        @pl.when(t == t_steps - 1)
        def _():
            out_ref[...] = (
                jnp.dot(h, wfc_ref[...], preferred_element_type=jnp.float32)
                + bfc_ref[...]
            )

    return pl.pallas_call(
        rnn_kernel,
        grid=(t_steps,),
        in_specs=[
            pl.BlockSpec((1, batch, emb), lambda t: (t, 0, 0)),
            pl.BlockSpec((emb, hid), lambda t: (0, 0)),
            pl.BlockSpec((hid, hid), lambda t: (0, 0)),
            pl.BlockSpec((1, hid), lambda t: (0, 0)),
            pl.BlockSpec((hid, out_dim), lambda t: (0, 0)),
            pl.BlockSpec((1, out_dim), lambda t: (0, 0)),
        ],
        out_specs=pl.BlockSpec((batch, out_dim), lambda t: (0, 0)),
        out_shape=jax.ShapeDtypeStruct((batch, out_dim), jnp.float32),
        scratch_shapes=[pltpu.VMEM((batch, hid), jnp.float32)],
        compiler_params=pltpu.CompilerParams(
            dimension_semantics=("arbitrary",),
        ),
    )


def kernel(x, embeddings, W_ih, W_hh, b_ih, b_hh, W_fc, b_fc):
    batch, t_steps = x.shape
    vocab, emb = embeddings.shape
    hid = W_ih.shape[0]
    out_dim = W_fc.shape[0]
    n_idx = batch * t_steps

    idx2d = x.T.reshape(n_idx // _GATHER_CHUNK, _GATHER_CHUNK)
    emb_flat = _make_sc_gather(vocab, emb, n_idx)(embeddings, idx2d)
    emb3 = emb_flat.reshape(t_steps, batch, emb)

    logits = _make_rnn_fc(t_steps, batch, emb, hid, out_dim)(
        emb3,
        W_ih.T,
        W_hh.T,
        (b_ih + b_hh).reshape(1, hid),
        W_fc.T,
        b_fc.reshape(1, out_dim),
    )
    return logits


# speedup vs baseline: 7.2702x; 1.7175x over previous
import functools

import jax
import jax.numpy as jnp
from jax import lax
from jax.experimental import pallas as pl
from jax.experimental.pallas import tpu as pltpu
from jax.experimental.pallas import tpu_sc as plsc

_NC = 2
_NS = 16
_NW = _NC * _NS
_CHUNK = 128
_NBUF = 5
_LOOKAHEAD = 3


@functools.lru_cache(maxsize=None)
def _make_sc_gather(vocab, emb, t_steps, batch):
    assert batch == _NW * _CHUNK
    assert t_steps % _NBUF == 0 and t_steps % 2 == 0
    n_groups = t_steps // _NBUF
    mesh = plsc.VectorSubcoreMesh(core_axis_name="c", subcore_axis_name="s")

    @functools.partial(
        pl.kernel,
        mesh=mesh,
        out_type=jax.ShapeDtypeStruct((t_steps // 2, batch, 2 * emb), jnp.float32),
        scratch_types=[
            pltpu.VMEM((t_steps, _CHUNK), jnp.int32),
            pltpu.VMEM((_NBUF, _CHUNK, emb), jnp.float32),
        ]
        + [pltpu.SemaphoreType.DMA] * (2 * _NBUF),
        compiler_params=pltpu.CompilerParams(use_tc_tiling_on_sc=False),
    )
    def gather_kernel(table_hbm, idx_hbm, out_hbm, idx_v, bufs, *sems):
        sem_g = sems[:_NBUF]
        sem_c = sems[_NBUF:]
        wid = lax.axis_index("s") * _NC + lax.axis_index("c")
        b0 = wid * _CHUNK

        pltpu.sync_copy(idx_hbm.at[:, wid], idx_v)

        def out_slice(t):
            return out_hbm.at[t // 2, pl.ds(b0, _CHUNK), pl.ds((t % 2) * emb, emb)]

        def gather_issue(t, b):
            pltpu.async_copy(table_hbm.at[idx_v.at[t]], bufs.at[b], sem_g[b])

        def gather_wait(t, b):
            pltpu.make_async_copy(
                table_hbm.at[idx_v.at[t]], bufs.at[b], sem_g[b]
            ).wait()

        def copyout_issue(t, b):
            pltpu.async_copy(bufs.at[b], out_slice(t), sem_c[b])

        def copyout_wait(t, b):
            pltpu.make_async_copy(bufs.at[b], out_slice(t), sem_c[b]).wait()

        for b in range(_LOOKAHEAD):
            gather_issue(b, b)

        def group(g, carry):
            for b in range(_NBUF):
                t = g * _NBUF + b
                gather_wait(t, b)
                copyout_issue(t, b)
                k = t + _LOOKAHEAD
                nb = (b + _LOOKAHEAD) % _NBUF

                @pl.when(k < t_steps)
                def _():
                    @pl.when(k >= _NBUF)
                    def _():
                        copyout_wait(k - _NBUF, nb)

                    gather_issue(k, nb)

            return carry

        lax.fori_loop(0, n_groups, group, 0)

        for b in range(_NBUF):
            copyout_wait(t_steps - _NBUF + b, b)

    return gather_kernel


@functools.lru_cache(maxsize=None)
def _make_rnn_fc(t_steps, batch, emb, hid, out_dim):
    n_pairs = t_steps // 2

    def rnn_kernel(emb_ref, we_ref, wo_ref, whh_ref, b_ref, wfc_ref, bfc_ref,
                   out_ref, h_ref):
        u = pl.program_id(0)

        @pl.when(u == 0)
        def _():
            h_ref[...] = jnp.zeros_like(h_ref)

        x2 = emb_ref[0]
        z_e = jnp.dot(x2, we_ref[...], preferred_element_type=jnp.float32)
        z_o = jnp.dot(x2, wo_ref[...], preferred_element_type=jnp.float32)
        h = jnp.tanh(
            z_e
            + jnp.dot(h_ref[...], whh_ref[...], preferred_element_type=jnp.float32)
            + b_ref[...]
        )
        h = jnp.tanh(
            z_o
            + jnp.dot(h, whh_ref[...], preferred_element_type=jnp.float32)
            + b_ref[...]
        )
        h_ref[...] = h

        @pl.when(u == n_pairs - 1)
        def _():
            out_ref[...] = (
                jnp.dot(h, wfc_ref[...], preferred_element_type=jnp.float32)
                + bfc_ref[...]
            )

    return pl.pallas_call(
        rnn_kernel,
        grid=(n_pairs,),
        in_specs=[
            pl.BlockSpec((1, batch, 2 * emb), lambda u: (u, 0, 0)),
            pl.BlockSpec((2 * emb, hid), lambda u: (0, 0)),
            pl.BlockSpec((2 * emb, hid), lambda u: (0, 0)),
            pl.BlockSpec((hid, hid), lambda u: (0, 0)),
            pl.BlockSpec((1, hid), lambda u: (0, 0)),
            pl.BlockSpec((hid, out_dim), lambda u: (0, 0)),
            pl.BlockSpec((1, out_dim), lambda u: (0, 0)),
        ],
        out_specs=pl.BlockSpec((batch, out_dim), lambda u: (0, 0)),
        out_shape=jax.ShapeDtypeStruct((batch, out_dim), jnp.float32),
        scratch_shapes=[pltpu.VMEM((batch, hid), jnp.float32)],
        compiler_params=pltpu.CompilerParams(
            dimension_semantics=("arbitrary",),
        ),
    )


def kernel(x, embeddings, W_ih, W_hh, b_ih, b_hh, W_fc, b_fc):
    batch, t_steps = x.shape
    vocab, emb = embeddings.shape
    hid = W_ih.shape[0]
    out_dim = W_fc.shape[0]

    idx3d = x.T.reshape(t_steps, _NW, _CHUNK)
    emb3 = _make_sc_gather(vocab, emb, t_steps, batch)(embeddings, idx3d)

    zeros = jnp.zeros((emb, hid), jnp.float32)
    w_even = jnp.concatenate([W_ih.T, zeros], axis=0)
    w_odd = jnp.concatenate([zeros, W_ih.T], axis=0)

    logits = _make_rnn_fc(t_steps, batch, emb, hid, out_dim)(
        emb3,
        w_even,
        w_odd,
        W_hh.T,
        (b_ih + b_hh).reshape(1, hid),
        W_fc.T,
        b_fc.reshape(1, out_dim),
    )
    return logits
